# Initial kernel scaffold; baseline (speedup 1.0000x reference)
import jax
import jax.numpy as jnp
from jax.experimental import pallas as pl


def kernel(inputs, indexes, features, labels):
    raise NotImplementedError("write your pallas kernel here")


# speedup vs baseline: 11.2588x; 11.2588x over previous
import functools

import jax
import jax.numpy as jnp
from jax import lax
from jax.experimental import pallas as pl
from jax.experimental.pallas import tpu as pltpu
from jax.experimental.pallas import tpu_sc as plsc

B = 1024
D = 128
NUM_MEMORY = 100000
NUM_CLASSES = 1000
TEMP = 0.05
EPS = 1e-06

C_PAD = 1024
CNT_W = 16

NC, NS = 2, 16
NW = NC * NS
RBLK = 128
NFULL = NUM_MEMORY // RBLK
REM = NUM_MEMORY - NFULL * RBLK
REM_TILE = NFULL % NW
MAXK = (NFULL + NW - 1) // NW
CROWS = C_PAD // NS


def _sc_body(feats, labels_h, zf, zc, ones_h,
             ps_out, cnt_out,
             feat_v, lab_v, featr_v, labr_v, ones_v, acc, csh):
    c = lax.axis_index("c")
    s = lax.axis_index("s")
    w = s * NC + c

    pltpu.sync_copy(ones_h, ones_v)
    pltpu.sync_copy(zf.at[pl.ds(s * CROWS, CROWS)], acc.at[pl.ds(s * CROWS, CROWS)])
    pltpu.sync_copy(zc.at[pl.ds(s * CROWS, CROWS)], csh.at[pl.ds(s * CROWS, CROWS)])
    plsc.subcore_barrier()

    def body(k, carry):
        blk = w + k * NW

        @pl.when(blk < NFULL)
        def _():
            st = pl.multiple_of(blk * RBLK, RBLK)
            pltpu.sync_copy(feats.at[pl.ds(st, RBLK)], feat_v)
            pltpu.sync_copy(labels_h.at[pl.ds(st, RBLK)], lab_v)
            pltpu.sync_copy(feat_v, acc.at[lab_v], add=True)
            pltpu.sync_copy(ones_v, csh.at[lab_v], add=True)

        return carry

    lax.fori_loop(0, MAXK, body, 0)

    @pl.when(w == REM_TILE)
    def _():
        st = NFULL * RBLK
        pltpu.sync_copy(feats.at[pl.ds(st, REM)], featr_v)
        pltpu.sync_copy(labels_h.at[pl.ds(st, REM)], labr_v)
        pltpu.sync_copy(featr_v, acc.at[labr_v], add=True)
        pltpu.sync_copy(ones_v.at[pl.ds(0, REM)], csh.at[labr_v], add=True)

    plsc.subcore_barrier()
    pltpu.sync_copy(acc.at[pl.ds(s * CROWS, CROWS)],
                    ps_out.at[c, pl.ds(s * CROWS, CROWS)])
    pltpu.sync_copy(csh.at[pl.ds(s * CROWS, CROWS)],
                    cnt_out.at[c, pl.ds(s * CROWS, CROWS)])


@functools.cache
def _sc_segsum():
  return pl.kernel(
    _sc_body,
    out_type=(
        jax.ShapeDtypeStruct((NC, C_PAD, D), jnp.float32),
        jax.ShapeDtypeStruct((NC, C_PAD, CNT_W), jnp.float32),
    ),
    mesh=plsc.VectorSubcoreMesh(core_axis_name="c", subcore_axis_name="s",
                                num_cores=NC, num_subcores=NS),
    scratch_types=[
        pltpu.VMEM((RBLK, D), jnp.float32),
        pltpu.VMEM((RBLK,), jnp.int32),
        pltpu.VMEM((REM, D), jnp.float32),
        pltpu.VMEM((REM,), jnp.int32),
        pltpu.VMEM((RBLK, CNT_W), jnp.float32),
        pltpu.VMEM_SHARED((C_PAD, D), jnp.float32),
        pltpu.VMEM_SHARED((C_PAD, CNT_W), jnp.float32),
    ],
  )


def _tc_body(x_ref, ps_ref, cnt_ref, tgt_ref, out_ref):
    x = x_ref[...]
    norm = jnp.sqrt(jnp.sum(x * x, axis=1, keepdims=True))
    x = x / jnp.maximum(norm, 1e-12)
    cf = ps_ref[0] + ps_ref[1]
    s = lax.dot_general(x, cf, dimension_numbers=(((1,), (1,)), ((), ())),
                        preferred_element_type=jnp.float32)
    nums = cnt_ref[0, 0:1, :] + cnt_ref[1, 0:1, :]
    mask = (nums > 0).astype(jnp.float32)
    denom = TEMP * (mask * nums + (1.0 - mask))
    sim = s / denom
    exps = jnp.exp(sim) * mask
    sums = jnp.sum(exps, axis=1, keepdims=True) + EPS
    t = tgt_ref[:, 0:1]
    cidx = lax.broadcasted_iota(jnp.int32, (B, C_PAD), 1)
    onehot = (cidx == t).astype(jnp.float32)
    picked = jnp.sum(exps * onehot, axis=1, keepdims=True)
    logp = jnp.log(picked / sums + EPS)
    out_ref[0, 0] = -jnp.mean(logp)


@functools.partial(jax.jit, static_argnames=("interpret",))
def _tc_loss(x, ps, cnt2, tgt2, interpret=False):
    return pl.pallas_call(
        _tc_body,
        out_shape=jax.ShapeDtypeStruct((1, 1), jnp.float32),
        out_specs=pl.BlockSpec(memory_space=pltpu.SMEM),
        interpret=interpret,
    )(x, ps, cnt2, tgt2)


def kernel(inputs, indexes, features, labels):
    targets = jnp.take(labels, indexes, axis=0)
    zf = jnp.zeros((C_PAD, D), jnp.float32)
    zc = jnp.zeros((C_PAD, CNT_W), jnp.float32)
    ones_h = jnp.ones((RBLK, CNT_W), jnp.float32)
    ps, cnt = _sc_segsum()(features, labels, zf, zc, ones_h)
    cnt2 = jnp.broadcast_to(cnt[:, :, 0][:, None, :], (NC, 8, C_PAD))
    tgt2 = jnp.broadcast_to(targets[:, None], (B, D)).astype(jnp.int32)
    out = _tc_loss(inputs, ps, cnt2, tgt2)
    return out[0, 0]


# speedup vs baseline: 17.3361x; 1.5398x over previous
import functools

import jax
import jax.numpy as jnp
from jax import lax
from jax.experimental import pallas as pl
from jax.experimental.pallas import tpu as pltpu
from jax.experimental.pallas import tpu_sc as plsc

B = 1024
D = 128
NUM_MEMORY = 100000
NUM_CLASSES = 1000
TEMP = 0.05
EPS = 1e-06

C_PAD = 1024
CNT_W = 16

NC, NS = 2, 16
NW = NC * NS
RBLK = 128
NFULL = NUM_MEMORY // RBLK
REM = NUM_MEMORY - NFULL * RBLK
REM_TILE = NFULL % NW
MAXK = (NFULL + NW - 1) // NW
CROWS = C_PAD // NS


def _sc_body(feats, labels_h, zf,
             ps_out, cnt_out,
             featA, featB, labA, labB, cnt_v, featr_v, labr_v,
             acc, semA, semB):
    c = lax.axis_index("c")
    s = lax.axis_index("s")
    w = s * NC + c

    pltpu.sync_copy(zf.at[pl.ds(s * CROWS, CROWS)], acc.at[pl.ds(s * CROWS, CROWS)])

    def zero_cnt(i, carry):
        cnt_v[pl.ds(i * 16, 16)] = jnp.zeros((16,), jnp.float32)
        return carry

    lax.fori_loop(0, C_PAD // 16, zero_cnt, 0)
    plsc.subcore_barrier()

    feat_bufs = (featA, featB)
    lab_bufs = (labA, labB)
    sems = (semA, semB)
    ones = jnp.full((16,), 1.0, jnp.float32)

    def start(k):
        blk = w + k * NW

        @pl.when(blk < NFULL)
        def _():
            st = pl.multiple_of(blk * RBLK, RBLK)
            pltpu.sync_copy(labels_h.at[pl.ds(st, RBLK)], lab_bufs[k % 2])
            pltpu.async_copy(feats.at[pl.ds(st, RBLK)], feat_bufs[k % 2], sems[k % 2])

    def count_from(lab_ref, nvec):
        for i in range(nvec):
            lv = lab_ref[pl.ds(i * 16, 16)]
            plsc.addupdate_scatter(cnt_v, [lv], ones)

    start(0)
    for k in range(MAXK):
        blk = w + k * NW
        if k + 1 < MAXK:
            start(k + 1)

        @pl.when(blk < NFULL)
        def _(k=k):
            pltpu.make_async_copy(feats.at[pl.ds(0, RBLK)],
                                  feat_bufs[k % 2], sems[k % 2]).wait()
            pltpu.sync_copy(feat_bufs[k % 2], acc.at[lab_bufs[k % 2]], add=True)
            count_from(lab_bufs[k % 2], RBLK // 16)

    @pl.when(w == REM_TILE)
    def _():
        st = NFULL * RBLK
        pltpu.sync_copy(feats.at[pl.ds(st, REM)], featr_v)
        pltpu.sync_copy(labels_h.at[pl.ds(st, REM)], labr_v)
        pltpu.sync_copy(featr_v, acc.at[labr_v], add=True)
        count_from(labr_v, REM // 16)

    pltpu.sync_copy(cnt_v, cnt_out.at[w])
    plsc.subcore_barrier()
    pltpu.sync_copy(acc.at[pl.ds(s * CROWS, CROWS)],
                    ps_out.at[c, pl.ds(s * CROWS, CROWS)])


@functools.cache
def _sc_segsum():
  return pl.kernel(
    _sc_body,
    out_type=(
        jax.ShapeDtypeStruct((NC, C_PAD, D), jnp.float32),
        jax.ShapeDtypeStruct((NW, C_PAD), jnp.float32),
    ),
    mesh=plsc.VectorSubcoreMesh(core_axis_name="c", subcore_axis_name="s",
                                num_cores=NC, num_subcores=NS),
    compiler_params=pltpu.CompilerParams(needs_layout_passes=False),
    scratch_types=[
        pltpu.VMEM((RBLK, D), jnp.float32),
        pltpu.VMEM((RBLK, D), jnp.float32),
        pltpu.VMEM((RBLK,), jnp.int32),
        pltpu.VMEM((RBLK,), jnp.int32),
        pltpu.VMEM((C_PAD,), jnp.float32),
        pltpu.VMEM((REM, D), jnp.float32),
        pltpu.VMEM((REM,), jnp.int32),
        pltpu.VMEM_SHARED((C_PAD, D), jnp.float32),
        pltpu.SemaphoreType.DMA,
        pltpu.SemaphoreType.DMA,
    ],
  )


def _tc_body(x_ref, ps_ref, cnt_ref, tgt_ref, out_ref):
    x = x_ref[...]
    norm = jnp.sqrt(jnp.sum(x * x, axis=1, keepdims=True))
    x = x / jnp.maximum(norm, 1e-12)
    cf = ps_ref[0] + ps_ref[1]
    s = lax.dot_general(x, cf, dimension_numbers=(((1,), (1,)), ((), ())),
                        preferred_element_type=jnp.float32)
    nums = jnp.sum(cnt_ref[...], axis=0, keepdims=True)
    mask = (nums > 0).astype(jnp.float32)
    denom = TEMP * (mask * nums + (1.0 - mask))
    sim = s / denom
    exps = jnp.exp(sim) * mask
    sums = jnp.sum(exps, axis=1, keepdims=True) + EPS
    t = tgt_ref[:, 0:1]
    cidx = lax.broadcasted_iota(jnp.int32, (B, C_PAD), 1)
    onehot = (cidx == t).astype(jnp.float32)
    picked = jnp.sum(exps * onehot, axis=1, keepdims=True)
    logp = jnp.log(picked / sums + EPS)
    out_ref[0, 0] = -jnp.mean(logp)


@functools.partial(jax.jit, static_argnames=("interpret",))
def _tc_loss(x, ps, cnt2, tgt2, interpret=False):
    return pl.pallas_call(
        _tc_body,
        out_shape=jax.ShapeDtypeStruct((1, 1), jnp.float32),
        out_specs=pl.BlockSpec(memory_space=pltpu.SMEM),
        interpret=interpret,
    )(x, ps, cnt2, tgt2)


def kernel(inputs, indexes, features, labels):
    targets = jnp.take(labels, indexes, axis=0)
    zf = jnp.zeros((C_PAD, D), jnp.float32)
    ps, cnt = _sc_segsum()(features, labels, zf)
    tgt2 = jnp.broadcast_to(targets[:, None], (B, D)).astype(jnp.int32)
    out = _tc_loss(inputs, ps, cnt, tgt2)
    return out[0, 0]


# speedup vs baseline: 17.8174x; 1.0278x over previous
import functools

import jax
import jax.numpy as jnp
from jax import lax
from jax.experimental import pallas as pl
from jax.experimental.pallas import tpu as pltpu
from jax.experimental.pallas import tpu_sc as plsc

B = 1024
D = 128
NUM_MEMORY = 100000
NUM_CLASSES = 1000
TEMP = 0.05
EPS = 1e-06

C_PAD = 1024
NC, NS = 2, 16
NW = NC * NS
RBLK = 128
CHUNK = 2 * RBLK
NCHUNK = NUM_MEMORY // CHUNK
EXTRA = NCHUNK % NW
BASEC = NCHUNK // NW
MAXJ = BASEC + 1
TAIL_START = NCHUNK * CHUNK
REM_START = TAIL_START + RBLK
REM = NUM_MEMORY - REM_START
LROWS = 2 * MAXJ
L2D = (NUM_MEMORY + RBLK - 1) // RBLK
TPW = B // NW
CROWS = C_PAD // NS
L2DP = 800
LSTAGE = 40


def _sc_body(feats, labels_h, labels2d, indexes_h, zf,
             ps_out, cnt_out, tgt_out,
             featA, featB, lab_all, cnt_v, idx_v, tgt_v, featr_v, labr_v,
             acc, semA, semB, semT):
    c = lax.axis_index("c")
    s = lax.axis_index("s")
    w = s * NC + c
    schunk = BASEC * w + jnp.minimum(w, EXTRA)
    nchunk = jnp.where(w < EXTRA, BASEC + 1, BASEC)

    pltpu.sync_copy(indexes_h.at[pl.ds(w * TPW, TPW)], idx_v)
    pltpu.async_copy(labels_h.at[idx_v], tgt_v, semT)

    schunk2 = schunk * 2
    base = pl.multiple_of((schunk2 // 8) * 8, 8)
    off = schunk2 - base
    pltpu.sync_copy(labels2d.at[pl.ds(base, LSTAGE)], lab_all)

    pltpu.sync_copy(zf.at[pl.ds(s * CROWS, CROWS)], acc.at[pl.ds(s * CROWS, CROWS)])

    def zero_cnt(i, carry):
        cnt_v[pl.ds(i * 16, 16)] = jnp.zeros((16,), jnp.float32)
        return carry

    lax.fori_loop(0, C_PAD // 16, zero_cnt, 0)
    plsc.subcore_barrier()

    feat_bufs = (featA, featB)
    sems = (semA, semB)
    ones = jnp.full((16,), 1.0, jnp.float32)

    def start(j):
        @pl.when(j < nchunk)
        def _():
            st = pl.multiple_of((schunk + j) * CHUNK, CHUNK)
            pltpu.async_copy(feats.at[pl.ds(st, CHUNK)], feat_bufs[j % 2], sems[j % 2])

    def count_row(row):
        for i in range(RBLK // 16):
            lv = lab_all[off + row, pl.ds(i * 16, 16)]
            plsc.addupdate_scatter(cnt_v, [lv], ones)

    start(0)
    for j in range(MAXJ):
        if j + 1 < MAXJ:
            start(j + 1)

        @pl.when(j < nchunk)
        def _(j=j):
            pltpu.make_async_copy(feats.at[pl.ds(0, CHUNK)],
                                  feat_bufs[j % 2], sems[j % 2]).wait()
            for r in range(2):
                pltpu.sync_copy(feat_bufs[j % 2].at[pl.ds(r * RBLK, RBLK)],
                                acc.at[lab_all.at[off + 2 * j + r]], add=True)
                count_row(2 * j + r)

    @pl.when(w == NW - 1)
    def _():
        pltpu.sync_copy(feats.at[pl.ds(TAIL_START, RBLK)], featA.at[pl.ds(0, RBLK)])
        pltpu.sync_copy(featA.at[pl.ds(0, RBLK)],
                        acc.at[lab_all.at[off + 2 * BASEC]], add=True)
        count_row(2 * BASEC)
        pltpu.sync_copy(feats.at[pl.ds(REM_START, REM)], featr_v)
        pltpu.sync_copy(labels_h.at[pl.ds(REM_START, REM)], labr_v)
        pltpu.sync_copy(featr_v, acc.at[labr_v], add=True)
        for i in range(REM // 16):
            lv = labr_v[pl.ds(i * 16, 16)]
            plsc.addupdate_scatter(cnt_v, [lv], ones)

    pltpu.sync_copy(cnt_v, cnt_out.at[pl.ds(w * C_PAD, C_PAD)])
    pltpu.make_async_copy(labels_h.at[pl.ds(0, TPW)], tgt_v, semT).wait()
    pltpu.sync_copy(tgt_v, tgt_out.at[pl.ds(w * TPW, TPW)])
    plsc.subcore_barrier()
    pltpu.sync_copy(acc.at[pl.ds(s * CROWS, CROWS)],
                    ps_out.at[c, pl.ds(s * CROWS, CROWS)])


@functools.cache
def _sc_segsum():
  return pl.kernel(
    _sc_body,
    out_type=(
        jax.ShapeDtypeStruct((NC, C_PAD, D), jnp.float32),
        jax.ShapeDtypeStruct((NW * C_PAD,), jnp.float32),
        jax.ShapeDtypeStruct((B,), jnp.int32),
    ),
    mesh=plsc.VectorSubcoreMesh(core_axis_name="c", subcore_axis_name="s",
                                num_cores=NC, num_subcores=NS),
    compiler_params=pltpu.CompilerParams(needs_layout_passes=False),
    scratch_types=[
        pltpu.VMEM((CHUNK, D), jnp.float32),
        pltpu.VMEM((CHUNK, D), jnp.float32),
        pltpu.VMEM((LSTAGE, RBLK), jnp.int32),
        pltpu.VMEM((C_PAD,), jnp.float32),
        pltpu.VMEM((TPW,), jnp.int32),
        pltpu.VMEM((TPW,), jnp.int32),
        pltpu.VMEM((REM, D), jnp.float32),
        pltpu.VMEM((REM,), jnp.int32),
        pltpu.VMEM_SHARED((C_PAD, D), jnp.float32),
        pltpu.SemaphoreType.DMA,
        pltpu.SemaphoreType.DMA,
        pltpu.SemaphoreType.DMA,
    ],
  )


def _tc_body(x_ref, ps_ref, cnt_ref, tgt_ref, out_ref):
    x = x_ref[...]
    norm = jnp.sqrt(jnp.sum(x * x, axis=1, keepdims=True))
    x = x / jnp.maximum(norm, 1e-12)
    cf = ps_ref[0] + ps_ref[1]
    s = lax.dot_general(x, cf, dimension_numbers=(((1,), (1,)), ((), ())),
                        preferred_element_type=jnp.float32)
    nums = jnp.sum(cnt_ref[...], axis=0, keepdims=True)
    mask = (nums > 0).astype(jnp.float32)
    denom = TEMP * (mask * nums + (1.0 - mask))
    sim = s / denom
    exps = jnp.exp(sim) * mask
    sums = jnp.sum(exps, axis=1, keepdims=True) + EPS
    t = tgt_ref[...]
    cidx = lax.broadcasted_iota(jnp.int32, (B, C_PAD), 1)
    onehot = (cidx == t).astype(jnp.float32)
    picked = jnp.sum(exps * onehot, axis=1, keepdims=True)
    logp = jnp.log(picked / sums + EPS)
    out_ref[0, 0] = -jnp.mean(logp)


@functools.partial(jax.jit, static_argnames=("interpret",))
def _tc_loss(x, ps, cnt2, tgt2, interpret=False):
    return pl.pallas_call(
        _tc_body,
        out_shape=jax.ShapeDtypeStruct((1, 1), jnp.float32),
        out_specs=pl.BlockSpec(memory_space=pltpu.SMEM),
        interpret=interpret,
    )(x, ps, cnt2, tgt2)


def kernel(inputs, indexes, features, labels):
    labels2d = jnp.pad(labels, (0, L2DP * RBLK - NUM_MEMORY)).reshape(L2DP, RBLK)
    zf = jnp.zeros((C_PAD, D), jnp.float32)
    ps, cnt, targets = _sc_segsum()(features, labels, labels2d,
                                    indexes.astype(jnp.int32), zf)
    out = _tc_loss(inputs, ps, cnt.reshape(NW, C_PAD), targets.reshape(B, 1))
    return out[0, 0]
